# baseline (speedup 1.0000x reference)
import functools

import jax
import jax.numpy as jnp
from jax import lax
from jax.experimental import pallas as pl
from jax.experimental.pallas import tpu as pltpu
from jax.experimental.pallas import tpu_sc as plsc

N, DIM = 8, 384
N_PAIR = N * DIM * 32
NC, NS = 2, 16
NW = NC * NS
PAIR_PER_W = N_PAIR // NW
CHUNK = 256
N_CHUNK = PAIR_PER_W // CHUNK

_MESH = plsc.VectorSubcoreMesh(
    core_axis_name="c", subcore_axis_name="s", num_cores=NC, num_subcores=NS
)


@functools.partial(
    pl.kernel,
    out_type=(
        jax.ShapeDtypeStruct((N_PAIR * 96,), jnp.float32),
        jax.ShapeDtypeStruct((N_PAIR * 32,), jnp.float32),
    ),
    mesh=_MESH,
    compiler_params=pltpu.CompilerParams(needs_layout_passes=False),
    scratch_types=[
        pltpu.VMEM((CHUNK * 128,), jnp.float32),
        pltpu.VMEM((CHUNK * 96,), jnp.float32),
        pltpu.VMEM((CHUNK * 32,), jnp.float32),
    ],
)
def _split_sc(x_hbm, z_hbm, c_hbm, xbuf, zbuf, cbuf):
    wid = lax.axis_index("s") * NC + lax.axis_index("c")
    ev2 = lax.iota(jnp.int32, 16) * 2

    def chunk_body(t, carry):
        base = wid * PAIR_PER_W + t * CHUNK
        pltpu.sync_copy(x_hbm.at[pl.ds(base * 128, CHUNK * 128)], xbuf)

        def pair_body(r, c2):
            src = pl.multiple_of(r * 128, 128)
            dz = pl.multiple_of(r * 96, 32)
            dc = pl.multiple_of(r * 32, 32)
            for k in range(4):
                zbuf[pl.ds(dz + 16 * k, 16)] = xbuf[pl.ds(src + 16 * k, 16)]
            for h in range(2):
                idx = ev2 + (src + 64 + 32 * h)
                zbuf[pl.ds(dz + 64 + 16 * h, 16)] = plsc.load_gather(
                    xbuf, [idx])
                cbuf[pl.ds(dc + 16 * h, 16)] = plsc.load_gather(
                    xbuf, [idx + 1])
            return c2

        lax.fori_loop(0, CHUNK, pair_body, 0)

        pltpu.sync_copy(zbuf, z_hbm.at[pl.ds(base * 96, CHUNK * 96)])
        pltpu.sync_copy(cbuf, c_hbm.at[pl.ds(base * 32, CHUNK * 32)])
        return carry

    lax.fori_loop(0, N_CHUNK, chunk_body, 0)


def kernel(x):
    xr = x.reshape(N_PAIR * 128)
    z1, c1 = _split_sc(xr)
    x_coarse = c1.reshape(N, DIM, 32, 32)
    z = z1.reshape(N, DIM, 3072)
    return (x_coarse, z)


# speedup vs baseline: 1.1521x; 1.1521x over previous
import functools

import jax
import jax.numpy as jnp
from jax import lax
from jax.experimental import pallas as pl
from jax.experimental.pallas import tpu as pltpu
from jax.experimental.pallas import tpu_sc as plsc

N, DIM = 8, 384
N_PAIR = N * DIM * 32
NC, NS = 2, 16
NW = NC * NS
PAIR_PER_W = N_PAIR // NW
CHUNK = 192
N_CHUNK = PAIR_PER_W // CHUNK

_MESH = plsc.VectorSubcoreMesh(
    core_axis_name="c", subcore_axis_name="s", num_cores=NC, num_subcores=NS
)

_BUF = lambda width: pltpu.VMEM((CHUNK * width,), jnp.float32)


@functools.partial(
    pl.kernel,
    out_type=(
        jax.ShapeDtypeStruct((N_PAIR * 96,), jnp.float32),
        jax.ShapeDtypeStruct((N_PAIR * 32,), jnp.float32),
    ),
    mesh=_MESH,
    compiler_params=pltpu.CompilerParams(needs_layout_passes=False),
    scratch_types=[
        _BUF(128), _BUF(128),
        _BUF(96), _BUF(96),
        _BUF(32), _BUF(32),
        pltpu.SemaphoreType.DMA, pltpu.SemaphoreType.DMA,
        pltpu.SemaphoreType.DMA, pltpu.SemaphoreType.DMA,
        pltpu.SemaphoreType.DMA, pltpu.SemaphoreType.DMA,
    ],
)
def _split_sc(x_hbm, z_hbm, c_hbm,
              xb0, xb1, zb0, zb1, cb0, cb1,
              ix0, ix1, sz0, sz1, sc0, sc1):
    wid = lax.axis_index("s") * NC + lax.axis_index("c")
    w0 = wid * PAIR_PER_W
    ev2 = lax.iota(jnp.int32, 16) * 2

    def in_copy(t, xb, sem):
        return pltpu.make_async_copy(
            x_hbm.at[pl.ds((w0 + t * CHUNK) * 128, CHUNK * 128)], xb, sem)

    def z_copy(t, zb, sem):
        return pltpu.make_async_copy(
            zb, z_hbm.at[pl.ds((w0 + t * CHUNK) * 96, CHUNK * 96)], sem)

    def c_copy(t, cb, sem):
        return pltpu.make_async_copy(
            cb, c_hbm.at[pl.ds((w0 + t * CHUNK) * 32, CHUNK * 32)], sem)

    def compute(xb, zb, cb):
        def pair_body(r, carry):
            src = pl.multiple_of(r * 128, 128)
            dz = pl.multiple_of(r * 96, 32)
            dc = pl.multiple_of(r * 32, 32)
            for k in range(4):
                zb[pl.ds(dz + 16 * k, 16)] = xb[pl.ds(src + 16 * k, 16)]
            for h in range(2):
                idx = ev2 + (src + 64 + 32 * h)
                zb[pl.ds(dz + 64 + 16 * h, 16)] = plsc.load_gather(xb, [idx])
                cb[pl.ds(dc + 16 * h, 16)] = plsc.load_gather(xb, [idx + 1])
            return carry

        lax.fori_loop(0, CHUNK, pair_body, 0, unroll=8)

    bufs = ((xb0, zb0, cb0, ix0, sz0, sc0), (xb1, zb1, cb1, ix1, sz1, sc1))

    in_copy(0, xb0, ix0).start()
    in_copy(1, xb1, ix1).start()

    def step(s, carry):
        for slot, (xb, zb, cb, ix, sz, sc) in enumerate(bufs):
            t = 2 * s + slot
            in_copy(t, xb, ix).wait()

            @pl.when(s > 0)
            def _():
                z_copy(t, zb, sz).wait()
                c_copy(t, cb, sc).wait()

            compute(xb, zb, cb)

            @pl.when(t + 2 < N_CHUNK)
            def _():
                in_copy(t + 2, xb, ix).start()

            z_copy(t, zb, sz).start()
            c_copy(t, cb, sc).start()
        return carry

    lax.fori_loop(0, N_CHUNK // 2, step, 0)

    for (xb, zb, cb, ix, sz, sc) in bufs:
        z_copy(0, zb, sz).wait()
        c_copy(0, cb, sc).wait()


def kernel(x):
    xr = x.reshape(N_PAIR * 128)
    z1, c1 = _split_sc(xr)
    x_coarse = c1.reshape(N, DIM, 32, 32)
    z = z1.reshape(N, DIM, 3072)
    return (x_coarse, z)


# speedup vs baseline: 1.2207x; 1.0596x over previous
import functools

import jax
import jax.numpy as jnp
from jax import lax
from jax.experimental import pallas as pl
from jax.experimental.pallas import tpu as pltpu
from jax.experimental.pallas import tpu_sc as plsc

N, DIM = 8, 384
N_PAIR = N * DIM * 32
NC, NS = 2, 16
NW = NC * NS
IMG_PER_W = N * DIM // NW
IMG_CHUNK = 6
CHUNK = IMG_CHUNK * 32
N_CHUNK = IMG_PER_W // IMG_CHUNK

_MESH = plsc.VectorSubcoreMesh(
    core_axis_name="c", subcore_axis_name="s", num_cores=NC, num_subcores=NS
)


@functools.partial(
    pl.kernel,
    out_type=(
        jax.ShapeDtypeStruct((N * DIM // 8, 24, 8, 128), jnp.float32),
        jax.ShapeDtypeStruct((N_PAIR * 32,), jnp.float32),
    ),
    mesh=_MESH,
    compiler_params=pltpu.CompilerParams(needs_layout_passes=False),
    scratch_types=[
        pltpu.VMEM((CHUNK, 128), jnp.float32),
        pltpu.VMEM((CHUNK, 128), jnp.float32),
        pltpu.VMEM((IMG_CHUNK, 24, 128), jnp.float32),
        pltpu.VMEM((IMG_CHUNK, 24, 128), jnp.float32),
        pltpu.VMEM((CHUNK * 32,), jnp.float32),
        pltpu.VMEM((CHUNK * 32,), jnp.float32),
        pltpu.SemaphoreType.DMA, pltpu.SemaphoreType.DMA,
        pltpu.SemaphoreType.DMA, pltpu.SemaphoreType.DMA,
        pltpu.SemaphoreType.DMA, pltpu.SemaphoreType.DMA,
    ],
)
def _split_sc(x_hbm, z_hbm, c_hbm,
              xb0, xb1, zb0, zb1, cb0, cb1,
              ix0, ix1, sz0, sz1, sc0, sc1):
    wid = lax.axis_index("s") * NC + lax.axis_index("c")
    w0 = wid * IMG_PER_W
    ev2 = lax.iota(jnp.int32, 16) * 2

    def in_copy(t, xb, sem):
        base = (w0 + t * IMG_CHUNK) * 32
        return pltpu.make_async_copy(x_hbm.at[pl.ds(base, CHUNK), :], xb, sem)

    def z_img_copy(t, i, zb, sem):
        gi = w0 + t * IMG_CHUNK + i
        return pltpu.make_async_copy(
            zb.at[i], z_hbm.at[gi // 8, :, gi % 8, :], sem)

    def c_copy(t, cb, sem):
        base = (w0 + t * IMG_CHUNK) * 32
        return pltpu.make_async_copy(
            cb, c_hbm.at[pl.ds(base * 32, CHUNK * 32)], sem)

    def compute(xb, zb, cb):
        def img_body(i, carry):
            q0 = i * 32
            for p in range(32):
                q = q0 + p
                row = jnp.full((16,), q, dtype=jnp.int32)
                dz = 96 * p
                for k in range(4):
                    off = dz + 16 * k
                    zb[i, off // 128, pl.ds(off % 128, 16)] = (
                        xb[q, pl.ds(16 * k, 16)])
                dc = pl.multiple_of(32 * q0 + 32 * p, 32)
                for h in range(2):
                    col = ev2 + (64 + 32 * h)
                    off = dz + 64 + 16 * h
                    zb[i, off // 128, pl.ds(off % 128, 16)] = (
                        plsc.load_gather(xb, [row, col]))
                    cb[pl.ds(dc + 16 * h, 16)] = (
                        plsc.load_gather(xb, [row, col + 1]))
            return carry

        lax.fori_loop(0, IMG_CHUNK, img_body, 0)

    bufs = ((xb0, zb0, cb0, ix0, sz0, sc0), (xb1, zb1, cb1, ix1, sz1, sc1))

    in_copy(0, xb0, ix0).start()
    in_copy(1, xb1, ix1).start()

    def step(s, carry):
        for slot, (xb, zb, cb, ix, sz, sc) in enumerate(bufs):
            t = 2 * s + slot
            in_copy(t, xb, ix).wait()

            @pl.when(s > 0)
            def _():
                for i in range(IMG_CHUNK):
                    z_img_copy(t, i, zb, sz).wait()
                c_copy(t, cb, sc).wait()

            compute(xb, zb, cb)

            @pl.when(t + 2 < N_CHUNK)
            def _():
                in_copy(t + 2, xb, ix).start()

            for i in range(IMG_CHUNK):
                z_img_copy(t, i, zb, sz).start()
            c_copy(t, cb, sc).start()
        return carry

    lax.fori_loop(0, N_CHUNK // 2, step, 0)

    for (xb, zb, cb, ix, sz, sc) in bufs:
        for i in range(IMG_CHUNK):
            z_img_copy(0, i, zb, sz).wait()
        c_copy(0, cb, sc).wait()


def kernel(x):
    x2 = x.reshape(N_PAIR, 128)
    z5, c1 = _split_sc(x2)
    x_coarse = c1.reshape(N, DIM, 32, 32)
    z = (z5.reshape(N, DIM // 8, 24, 8, 128)
         .transpose(0, 1, 3, 2, 4)
         .reshape(N, DIM, 3072))
    return (x_coarse, z)


# speedup vs baseline: 1.9369x; 1.5867x over previous
import functools

import jax
import jax.numpy as jnp
from jax import lax
from jax.experimental import pallas as pl
from jax.experimental.pallas import tpu as pltpu
from jax.experimental.pallas import tpu_sc as plsc

N, DIM = 8, 384
N_IMG = N * DIM
NC, NS = 2, 16
NW = NC * NS
IMG_PER_W = N_IMG // NW
IMG_CHUNK = 4
N_CHUNK = IMG_PER_W // IMG_CHUNK
ROWS = IMG_CHUNK * 64

_MESH = plsc.VectorSubcoreMesh(
    core_axis_name="c", subcore_axis_name="s", num_cores=NC, num_subcores=NS
)


@functools.partial(
    pl.kernel,
    out_type=(
        jax.ShapeDtypeStruct((N_IMG, 3072), jnp.float32),
        jax.ShapeDtypeStruct((N_IMG * 32, 32), jnp.float32),
    ),
    mesh=_MESH,
    compiler_params=pltpu.CompilerParams(needs_layout_passes=False),
    scratch_types=[
        pltpu.VMEM((ROWS, 64), jnp.float32),
        pltpu.VMEM((ROWS, 64), jnp.float32),
        pltpu.VMEM((IMG_CHUNK * 3072,), jnp.float32),
        pltpu.VMEM((IMG_CHUNK * 3072,), jnp.float32),
        pltpu.VMEM((IMG_CHUNK * 32, 32), jnp.float32),
        pltpu.VMEM((IMG_CHUNK * 32, 32), jnp.float32),
        pltpu.SemaphoreType.DMA, pltpu.SemaphoreType.DMA,
        pltpu.SemaphoreType.DMA, pltpu.SemaphoreType.DMA,
        pltpu.SemaphoreType.DMA, pltpu.SemaphoreType.DMA,
    ],
)
def _split_sc(x_hbm, z_hbm, c_hbm,
              xb0, xb1, zb0, zb1, cb0, cb1,
              ix0, ix1, sz0, sz1, sc0, sc1):
    wid = lax.axis_index("s") * NC + lax.axis_index("c")
    w0 = wid * IMG_PER_W
    ev2 = lax.iota(jnp.int32, 16) * 2

    def in_copy(t, xb, sem):
        gi0 = w0 + t * IMG_CHUNK
        return pltpu.make_async_copy(
            x_hbm.at[pl.ds(gi0 * 64, ROWS), :], xb, sem)

    def z_img_copy(t, i, zb, sem):
        gi = w0 + t * IMG_CHUNK + i
        return pltpu.make_async_copy(
            zb.at[pl.ds(i * 3072, 3072)], z_hbm.at[gi], sem)

    def c_copy(t, cb, sem):
        gi0 = w0 + t * IMG_CHUNK
        return pltpu.make_async_copy(
            cb, c_hbm.at[pl.ds(gi0 * 32, IMG_CHUNK * 32), :], sem)

    def compute(xb, zb, cb):
        def img_body(i, carry):
            r0 = i * 64
            q0 = i * 32
            for p in range(32):
                le = r0 + 2 * p
                row = jnp.full((16,), le + 1, dtype=jnp.int32)
                dz = i * 3072 + 96 * p
                for k in range(4):
                    zb[pl.ds(dz + 16 * k, 16)] = xb[le, pl.ds(16 * k, 16)]
                for h in range(2):
                    col = ev2 + 32 * h
                    zb[pl.ds(dz + 64 + 16 * h, 16)] = (
                        plsc.load_gather(xb, [row, col]))
                    cb[q0 + p, pl.ds(16 * h, 16)] = (
                        plsc.load_gather(xb, [row, col + 1]))
            return carry

        lax.fori_loop(0, IMG_CHUNK, img_body, 0)

    bufs = ((xb0, zb0, cb0, ix0, sz0, sc0), (xb1, zb1, cb1, ix1, sz1, sc1))

    in_copy(0, xb0, ix0).start()
    in_copy(1, xb1, ix1).start()

    def step(s, carry):
        for slot, (xb, zb, cb, ix, sz, sc) in enumerate(bufs):
            t = 2 * s + slot
            in_copy(t, xb, ix).wait()

            @pl.when(s > 0)
            def _():
                for i in range(IMG_CHUNK):
                    z_img_copy(t, i, zb, sz).wait()
                c_copy(t, cb, sc).wait()

            compute(xb, zb, cb)

            @pl.when(t + 2 < N_CHUNK)
            def _():
                in_copy(t + 2, xb, ix).start()

            for i in range(IMG_CHUNK):
                z_img_copy(t, i, zb, sz).start()
            c_copy(t, cb, sc).start()
        return carry

    lax.fori_loop(0, N_CHUNK // 2, step, 0)

    for (xb, zb, cb, ix, sz, sc) in bufs:
        for i in range(IMG_CHUNK):
            z_img_copy(0, i, zb, sz).wait()
        c_copy(0, cb, sc).wait()


def kernel(x):
    x2 = x.reshape(N_IMG * 64, 64)
    z2, c2 = _split_sc(x2)
    x_coarse = c2.reshape(N, DIM, 32, 32)
    z = z2.reshape(N, DIM, 3072)
    return (x_coarse, z)
